# baseline (speedup 1.0000x reference)
import functools

import jax
import jax.numpy as jnp
from jax import lax
from jax.experimental import pallas as pl
from jax.experimental.pallas import tpu as pltpu
from jax.experimental.pallas import tpu_sc as plsc

_NBUF = 6
_AHEAD = 4
_UNROLL = 8


def kernel(x, emb_weight):
    B, S, D = x.shape
    info = plsc.get_sparse_core_info()
    NC, NS = info.num_cores, info.num_subcores
    NW = NC * NS
    ROWS = S // NW

    mesh = plsc.VectorSubcoreMesh(core_axis_name="c", subcore_axis_name="s")

    @functools.partial(
        pl.kernel,
        mesh=mesh,
        out_type=jax.ShapeDtypeStruct((B, S, D), jnp.float32),
        scratch_types=(
            [pltpu.VMEM((ROWS, D), jnp.float32)]
            + [pltpu.VMEM((1, ROWS, D), jnp.float32)] * _NBUF
            + [pltpu.SemaphoreType.DMA] * (2 * _NBUF + 1)
        ),
        compiler_params=pltpu.CompilerParams(use_tc_tiling_on_sc=True),
    )
    def run(x_hbm, emb_hbm, out_hbm, emb_v, *rest):
        buf = rest[:_NBUF]
        load_sems = rest[_NBUF : 2 * _NBUF]
        store_sems = rest[2 * _NBUF : 3 * _NBUF]
        emb_sem = rest[3 * _NBUF]
        wid = lax.axis_index("s") * NC + lax.axis_index("c")
        r0 = wid * ROWS

        emb_h = pltpu.async_copy(emb_hbm.at[pl.ds(r0, ROWS), :], emb_v, emb_sem)

        load_h = {}
        store_h = {}
        for b in range(min(_AHEAD, B)):
            load_h[b] = pltpu.async_copy(
                x_hbm.at[pl.ds(b, 1), pl.ds(r0, ROWS), :], buf[b], load_sems[b]
            )
        emb_h.wait()
        for b in range(B):
            k = b % _NBUF
            load_h[b].wait()
            bb = buf[k]

            d_shift = D.bit_length() - 1

            @plsc.parallel_loop(0, ROWS * D, step=16, unroll=_UNROLL)
            def body(off, bb=bb, d_shift=d_shift):
                r = off >> d_shift
                c = pl.multiple_of(off & (D - 1), 16)
                sl = pl.ds(c, 16)
                plsc.addupdate(bb.at[0, r, sl], emb_v[r, sl])

            store_h[b] = pltpu.async_copy(
                bb, out_hbm.at[pl.ds(b, 1), pl.ds(r0, ROWS), :], store_sems[k]
            )
            nb = b + _AHEAD
            if nb < B:
                if nb - _NBUF >= 0:
                    store_h[nb - _NBUF].wait()
                load_h[nb] = pltpu.async_copy(
                    x_hbm.at[pl.ds(nb, 1), pl.ds(r0, ROWS), :],
                    buf[nb % _NBUF],
                    load_sems[nb % _NBUF],
                )
        for b in range(B - _NBUF, B):
            store_h[b].wait()
    return run(x, emb_weight)
